# baseline (speedup 1.0000x reference)
import functools

import jax
import jax.numpy as jnp
from jax import lax
from jax.experimental import pallas as pl
from jax.experimental.pallas import tpu as pltpu
from jax.experimental.pallas import tpu_sc as plsc

EMB_DIM = 16
NUM_WORKERS = 32
ROWS = 16384 * 26
ROWS_PER_W = ROWS // NUM_WORKERS
CHUNK = 1664
NCHUNK = ROWS_PER_W // CHUNK
MAGIC = 12582912.0
QLOW = -128.0
QHIGH = 127.0

_mesh = plsc.VectorSubcoreMesh(core_axis_name="c", subcore_axis_name="s")


@functools.partial(
    pl.kernel,
    out_type=jax.ShapeDtypeStruct((ROWS, EMB_DIM), jnp.float32),
    mesh=_mesh,
    scratch_types=[
        pltpu.VMEM((ROWS_PER_W,), jnp.int32),
        pltpu.VMEM((2, CHUNK, EMB_DIM), jnp.float32),
        pltpu.VMEM((2, 16), jnp.float32),
        pltpu.SemaphoreType.DMA,
        pltpu.SemaphoreType.DMA,
    ],
    compiler_params=pltpu.CompilerParams(use_tc_tiling_on_sc=False),
)
def _lsq_lookup(idx_hbm, w_hbm, scale_hbm, out_hbm,
                idx_v, rows_v, scale_v, sem0, sem1):
    wid = lax.axis_index("s") * 2 + lax.axis_index("c")
    base = wid * ROWS_PER_W

    pltpu.sync_copy(scale_hbm, scale_v)
    pltpu.sync_copy(idx_hbm.at[pl.ds(base, ROWS_PER_W)], idx_v)
    inv_a = scale_v[0, :]
    a = scale_v[1, :]

    sems = (sem0, sem1)
    copies = [None, None]
    copies[0] = pltpu.async_copy(
        w_hbm.at[idx_v.at[pl.ds(0, CHUNK)]], rows_v.at[0], sems[0])

    for c in range(NCHUNK):
        buf = c % 2
        if c + 1 < NCHUNK:
            copies[1 - buf] = pltpu.async_copy(
                w_hbm.at[idx_v.at[pl.ds((c + 1) * CHUNK, CHUNK)]],
                rows_v.at[1 - buf], sems[1 - buf])
        copies[buf].wait()

        def body(i, _):
            y = rows_v[buf, i, :] * inv_a
            r = (y + MAGIC) - MAGIC
            r = jnp.minimum(jnp.maximum(r, QLOW), QHIGH)
            rows_v[buf, i, :] = r * a
            return 0

        lax.fori_loop(0, CHUNK, body, 0, unroll=8)

        pltpu.sync_copy(rows_v.at[buf],
                        out_hbm.at[pl.ds(base + c * CHUNK, CHUNK)])


def kernel(x, weight, alpha):
    a = jnp.abs(alpha).astype(jnp.float32) + 1e-10
    scale = jnp.stack([jnp.full((16,), 1.0 / a, jnp.float32),
                       jnp.full((16,), a, jnp.float32)])
    idx = x.reshape(-1).astype(jnp.int32)
    out = _lsq_lookup(idx, weight, scale)
    return out.reshape(x.shape + (EMB_DIM,))


# speedup vs baseline: 1.0449x; 1.0449x over previous
import functools

import jax
import jax.numpy as jnp
from jax import lax
from jax.experimental import pallas as pl
from jax.experimental.pallas import tpu as pltpu
from jax.experimental.pallas import tpu_sc as plsc

EMB_DIM = 16
LANES = 128
SUP_ROWS = 1000000 * EMB_DIM // LANES
ROWS = 16384 * 26
NUM_WORKERS = 32
RPW = ROWS // NUM_WORKERS
CHUNK = 256
NCHUNK = RPW // CHUNK
NPAIR = NCHUNK // 2
C8 = CHUNK // 8
OUT_RPW = RPW // 8
MAGIC = 12582912.0
QLOW = -128.0
QHIGH = 127.0

_mesh = plsc.VectorSubcoreMesh(core_axis_name="c", subcore_axis_name="s")


@functools.partial(
    pl.kernel,
    out_type=jax.ShapeDtypeStruct((ROWS // 8, LANES), jnp.float32),
    mesh=_mesh,
    scratch_types=[
        pltpu.VMEM((RPW,), jnp.int32),
        pltpu.VMEM((2, CHUNK // 128, 128), jnp.int32),
        pltpu.VMEM((2, CHUNK), jnp.int32),
        pltpu.VMEM((2, CHUNK, LANES), jnp.float32),
        pltpu.VMEM((2, C8, LANES), jnp.float32),
        pltpu.VMEM((32,), jnp.float32),
        pltpu.SemaphoreType.DMA,
        pltpu.SemaphoreType.DMA,
        pltpu.SemaphoreType.DMA,
        pltpu.SemaphoreType.DMA,
    ],
    compiler_params=pltpu.CompilerParams(needs_layout_passes=False),
)
def _lsq_lookup(idx_hbm, w_hbm, scale_hbm, out_hbm,
                idx_v, sup_v, sub_v, super_v, dst_v, scale_v,
                gsem0, gsem1, osem0, osem1):
    wid = lax.axis_index("s") * 2 + lax.axis_index("c")
    base = wid * RPW
    obase = wid * OUT_RPW

    pltpu.sync_copy(scale_hbm, scale_v)
    pltpu.sync_copy(idx_hbm.at[pl.ds(base, RPW)], idx_v)
    inv_a = scale_v[pl.ds(0, 16)]
    a = scale_v[pl.ds(16, 16)]
    iota = lax.iota(jnp.int32, 16)

    gsems = (gsem0, gsem1)
    osems = (osem0, osem1)

    def supsub(c, slot):
        for j in range(CHUNK // 128):
            def body(i, _):
                v = idx_v[pl.ds(c * CHUNK + j * 128 + i * 16, 16)]
                sup_v[slot, j, pl.ds(i * 16, 16)] = (
                    lax.shift_right_logical(v, 3))
                sub_v[slot, pl.ds(j * 128 + i * 16, 16)] = (v & 7) * 16
                return 0
            lax.fori_loop(0, 8, body, 0, unroll=4)

    def start_gather(slot):
        for j in range(CHUNK // 128):
            pltpu.async_copy(
                w_hbm.at[sup_v.at[slot, j]],
                super_v.at[slot, pl.ds(j * 128, 128)], gsems[slot])

    def wait_gather(slot):
        for j in range(CHUNK // 128):
            pltpu.make_async_copy(
                w_hbm.at[sup_v.at[slot, j]],
                super_v.at[slot, pl.ds(j * 128, 128)], gsems[slot]).wait()

    def start_out(c, slot):
        return pltpu.async_copy(
            dst_v.at[slot], out_hbm.at[pl.ds(obase + c * C8, C8)],
            osems[slot])

    def wait_out(c, slot):
        pltpu.make_async_copy(
            dst_v.at[slot], out_hbm.at[pl.ds(obase + c * C8, C8)],
            osems[slot]).wait()

    def extract_quant(slot):
        def body(g2, _):
            local = g2 * 16 + iota
            s16 = sub_v[slot, pl.ds(g2 * 16, 16)]
            orow = lax.shift_right_logical(local, 3)
            ocol0 = (local & 7) * 16
            for k in range(16):
                v = plsc.load_gather(super_v.at[slot], [local, s16 + k])
                r = (v * inv_a + MAGIC) - MAGIC
                r = jnp.minimum(jnp.maximum(r, QLOW), QHIGH)
                plsc.store_scatter(dst_v.at[slot], [orow, ocol0 + k], r * a)
            return 0
        lax.fori_loop(0, CHUNK // 16, body, 0)

    supsub(0, 0)
    start_gather(0)
    supsub(1, 1)
    start_gather(1)

    def pair(p, _):
        for slot in (0, 1):
            c = 2 * p + slot

            @pl.when(p >= 1)
            def _():
                wait_out(c - 2, slot)

            wait_gather(slot)
            extract_quant(slot)
            start_out(c, slot)

            @pl.when(p < NPAIR - 1)
            def _():
                supsub(c + 2, slot)
                start_gather(slot)
        return 0

    lax.fori_loop(0, NPAIR, pair, 0)
    wait_out(NCHUNK - 2, 0)
    wait_out(NCHUNK - 1, 1)


def kernel(x, weight, alpha):
    a = jnp.abs(alpha).astype(jnp.float32) + 1e-10
    scale = jnp.concatenate([jnp.full((16,), 1.0 / a, jnp.float32),
                             jnp.full((16,), a, jnp.float32)])
    idx = x.reshape(-1).astype(jnp.int32)
    w2 = weight.reshape(SUP_ROWS, LANES)
    out = _lsq_lookup(idx, w2, scale)
    return out.reshape(x.shape + (EMB_DIM,))


# speedup vs baseline: 1.4356x; 1.3739x over previous
import functools

import jax
import jax.numpy as jnp
from jax import lax
from jax.experimental import pallas as pl
from jax.experimental.pallas import tpu as pltpu
from jax.experimental.pallas import tpu_sc as plsc

EMB_DIM = 16
VOCAB = 1000000
SUP_ROWS = VOCAB * EMB_DIM // 128
BATCH = 16384
SLOTS = 26
NUM_WORKERS = 32
NBLK = BATCH // 128
BPW = NBLK // NUM_WORKERS
TILES_PW = SLOTS * BPW
MAGIC = 12582912.0
QLOW = -128.0
QHIGH = 127.0

_mesh = plsc.VectorSubcoreMesh(core_axis_name="c", subcore_axis_name="s")


@functools.partial(
    pl.kernel,
    out_type=jax.ShapeDtypeStruct((SLOTS, 2, NBLK, 8, 128), jnp.float32),
    mesh=_mesh,
    scratch_types=[
        pltpu.VMEM((SLOTS, BPW, 128), jnp.int32),
        pltpu.VMEM((2, 128), jnp.int32),
        pltpu.VMEM((2, 128), jnp.int32),
        pltpu.VMEM((2, 128, 128), jnp.float32),
        pltpu.VMEM((2, EMB_DIM, 128), jnp.float32),
        pltpu.VMEM((32,), jnp.float32),
        pltpu.SemaphoreType.DMA,
        pltpu.SemaphoreType.DMA,
        pltpu.SemaphoreType.DMA,
        pltpu.SemaphoreType.DMA,
        pltpu.SemaphoreType.DMA,
    ],
    compiler_params=pltpu.CompilerParams(needs_layout_passes=False),
)
def _lsq_lookup(xt_hbm, w_hbm, scale_hbm, out_hbm,
                idx_v, sup_v, sub_v, super_v, res_v, scale_v,
                isem, gsem0, gsem1, osem0, osem1):
    wid = lax.axis_index("s") * 2 + lax.axis_index("c")
    col0 = wid * (BPW * 128)

    pltpu.sync_copy(scale_hbm, scale_v)
    for j in range(BPW):
        pltpu.async_copy(
            xt_hbm.at[:, pl.ds(col0 + j * 128, 128)], idx_v.at[:, j, :],
            isem)
    for j in range(BPW):
        pltpu.make_async_copy(
            xt_hbm.at[:, pl.ds(col0 + j * 128, 128)], idx_v.at[:, j, :],
            isem).wait()
    inv_a = scale_v[pl.ds(0, 16)]
    a = scale_v[pl.ds(16, 16)]

    gsems = (gsem0, gsem1)
    osems = (osem0, osem1)

    def supsub(t, slot):
        s = lax.div(t, BPW)
        j = lax.rem(t, BPW)

        def body(i, _):
            v = idx_v[s, j, pl.ds(i * 16, 16)]
            sup_v[slot, pl.ds(i * 16, 16)] = lax.shift_right_logical(v, 3)
            sub_v[slot, pl.ds(i * 16, 16)] = (v & 7) * 16
            return 0
        lax.fori_loop(0, 8, body, 0, unroll=4)

    def start_gather(slot):
        pltpu.async_copy(
            w_hbm.at[sup_v.at[slot]], super_v.at[slot], gsems[slot])

    def wait_gather(slot):
        pltpu.make_async_copy(
            w_hbm.at[sup_v.at[slot]], super_v.at[slot], gsems[slot]).wait()

    def start_out(t, slot):
        s = lax.div(t, BPW)
        j = lax.rem(t, BPW)
        for eb in range(2):
            pltpu.async_copy(
                res_v.at[slot, pl.ds(eb * 8, 8)],
                out_hbm.at[s, eb, wid * BPW + j], osems[slot])

    def wait_out(t, slot):
        s = lax.div(t, BPW)
        j = lax.rem(t, BPW)
        for eb in range(2):
            pltpu.make_async_copy(
                res_v.at[slot, pl.ds(eb * 8, 8)],
                out_hbm.at[s, eb, wid * BPW + j], osems[slot]).wait()

    iota = lax.iota(jnp.int32, 16)

    def extract_quant(slot):
        def body(i16, _):
            local = i16 * 16 + iota
            s16 = sub_v[slot, pl.ds(i16 * 16, 16)]
            for k in range(EMB_DIM):
                v = plsc.load_gather(super_v.at[slot], [local, s16 + k])
                r = (v * inv_a + MAGIC) - MAGIC
                r = jnp.minimum(jnp.maximum(r, QLOW), QHIGH)
                res_v[slot, k, pl.ds(i16 * 16, 16)] = r * a
            return 0
        lax.fori_loop(0, 8, body, 0)

    supsub(0, 0)
    start_gather(0)
    supsub(1, 1)
    start_gather(1)

    def pair(p, _):
        for slot in (0, 1):
            t = 2 * p + slot
            wait_gather(slot)

            @pl.when(p >= 1)
            def _():
                wait_out(t - 2, slot)

            extract_quant(slot)

            @pl.when(p < TILES_PW // 2 - 1)
            def _():
                supsub(t + 2, slot)
                start_gather(slot)

            start_out(t, slot)
        return 0

    lax.fori_loop(0, TILES_PW // 2, pair, 0)
    wait_out(TILES_PW - 2, 0)
    wait_out(TILES_PW - 1, 1)


def kernel(x, weight, alpha):
    a = jnp.abs(alpha).astype(jnp.float32) + 1e-10
    scale = jnp.concatenate([jnp.full((16,), 1.0 / a, jnp.float32),
                             jnp.full((16,), a, jnp.float32)])
    xt = x.T.astype(jnp.int32)
    w2 = weight.reshape(SUP_ROWS, 128)
    out5 = _lsq_lookup(xt, w2, scale)
    return out5.transpose(2, 4, 0, 1, 3).reshape(BATCH, SLOTS, EMB_DIM)


# speedup vs baseline: 1.5479x; 1.0783x over previous
import functools

import jax
import jax.numpy as jnp
from jax import lax
from jax.experimental import pallas as pl
from jax.experimental.pallas import tpu as pltpu
from jax.experimental.pallas import tpu_sc as plsc

EMB_DIM = 16
VOCAB = 1000000
SUP_ROWS = VOCAB * EMB_DIM // 128
BATCH = 16384
SLOTS = 26
NUM_WORKERS = 32
NBLK = BATCH // 128
BPW = NBLK // NUM_WORKERS
TILES_PW = SLOTS * BPW
MAGIC = 12582912.0
QLOW = -128.0
QHIGH = 127.0

_mesh = plsc.VectorSubcoreMesh(core_axis_name="c", subcore_axis_name="s")

NTILE_T = VOCAB // 128
TAIL_COLS = VOCAB - NTILE_T * 128
TITER = (NTILE_T + NUM_WORKERS - 1) // NUM_WORKERS


@functools.partial(
    pl.kernel,
    out_type=jax.ShapeDtypeStruct((SUP_ROWS, 128), jnp.float32),
    mesh=_mesh,
    scratch_types=[
        pltpu.VMEM((2, EMB_DIM, 128), jnp.float32),
        pltpu.VMEM((2, EMB_DIM, 128), jnp.float32),
        pltpu.VMEM((EMB_DIM, TAIL_COLS), jnp.float32),
        pltpu.SemaphoreType.DMA,
        pltpu.SemaphoreType.DMA,
        pltpu.SemaphoreType.DMA,
        pltpu.SemaphoreType.DMA,
    ],
    compiler_params=pltpu.CompilerParams(needs_layout_passes=False),
)
def _transpose_table(wt_hbm, w2_hbm, tin_v, tout_v, tail_v, l0, l1, s0, s1):
    wid = lax.axis_index("s") * 2 + lax.axis_index("c")
    iota = lax.iota(jnp.int32, 16)
    lsems = (l0, l1)
    ssems = (s0, s1)

    def c0_of(k):
        return pl.multiple_of((wid + NUM_WORKERS * k) * 128, 128)

    def sp0_of(k):
        return pl.multiple_of((wid + NUM_WORKERS * k) * 16, 16)

    def valid(k):
        return (wid + NUM_WORKERS * k) < NTILE_T

    def start_load(k, slot):
        pltpu.async_copy(wt_hbm.at[:, pl.ds(c0_of(k), 128)],
                         tin_v.at[slot], lsems[slot])

    def wait_load(k, slot):
        pltpu.make_async_copy(wt_hbm.at[:, pl.ds(c0_of(k), 128)],
                              tin_v.at[slot], lsems[slot]).wait()

    def start_store(k, slot):
        pltpu.async_copy(tout_v.at[slot],
                         w2_hbm.at[pl.ds(sp0_of(k), 16)], ssems[slot])

    def wait_store(k, slot):
        pltpu.make_async_copy(tout_v.at[slot],
                              w2_hbm.at[pl.ds(sp0_of(k), 16)],
                              ssems[slot]).wait()

    @pl.when(valid(0))
    def _():
        start_load(0, 0)

    @pl.when(valid(1))
    def _():
        start_load(1, 1)

    def pair(p, _):
        for slot in (0, 1):
            k = 2 * p + slot

            @pl.when(valid(k))
            def _():
                wait_load(k, slot)

                @pl.when(k >= 2)
                def _():
                    wait_store(k - 2, slot)

                for sp in range(16):
                    for seg in range(8):
                        col = jnp.full((16,), sp * 8 + seg, jnp.int32)
                        v = plsc.load_gather(tin_v.at[slot], [iota, col])
                        tout_v[slot, sp, pl.ds(seg * 16, 16)] = v

                @pl.when(valid(k + 2))
                def _():
                    start_load(k + 2, slot)

                start_store(k, slot)
        return 0

    lax.fori_loop(0, (TITER + 1) // 2, pair, 0)

    @pl.when(valid(TITER - 2))
    def _():
        wait_store(TITER - 2, (TITER - 2) % 2)

    @pl.when(valid(TITER - 1))
    def _():
        wait_store(TITER - 1, (TITER - 1) % 2)

    @pl.when(wid == 0)
    def _():
        pltpu.sync_copy(wt_hbm.at[:, pl.ds(NTILE_T * 128, TAIL_COLS)],
                        tail_v)
        for sp in range(TAIL_COLS // 8):
            for seg in range(8):
                col = jnp.full((16,), sp * 8 + seg, jnp.int32)
                v = plsc.load_gather(tail_v, [iota, col])
                tout_v[0, sp, pl.ds(seg * 16, 16)] = v
        pltpu.sync_copy(tout_v.at[0, pl.ds(0, TAIL_COLS // 8)],
                        w2_hbm.at[pl.ds(NTILE_T * 16, TAIL_COLS // 8)])


@functools.partial(
    pl.kernel,
    out_type=jax.ShapeDtypeStruct((SLOTS, 2, NBLK, 8, 128), jnp.float32),
    mesh=_mesh,
    scratch_types=[
        pltpu.VMEM((SLOTS, BPW, 128), jnp.int32),
        pltpu.VMEM((2, 128), jnp.int32),
        pltpu.VMEM((2, 128), jnp.int32),
        pltpu.VMEM((2, 128, 128), jnp.float32),
        pltpu.VMEM((2, EMB_DIM, 128), jnp.float32),
        pltpu.VMEM((32,), jnp.float32),
        pltpu.SemaphoreType.DMA,
        pltpu.SemaphoreType.DMA,
        pltpu.SemaphoreType.DMA,
        pltpu.SemaphoreType.DMA,
        pltpu.SemaphoreType.DMA,
    ],
    compiler_params=pltpu.CompilerParams(needs_layout_passes=False),
)
def _lsq_lookup(xt_hbm, w_hbm, scale_hbm, out_hbm,
                idx_v, sup_v, sub_v, super_v, res_v, scale_v,
                isem, gsem0, gsem1, osem0, osem1):
    wid = lax.axis_index("s") * 2 + lax.axis_index("c")
    col0 = wid * (BPW * 128)

    pltpu.sync_copy(scale_hbm, scale_v)
    for j in range(BPW):
        pltpu.async_copy(
            xt_hbm.at[:, pl.ds(col0 + j * 128, 128)], idx_v.at[:, j, :],
            isem)
    for j in range(BPW):
        pltpu.make_async_copy(
            xt_hbm.at[:, pl.ds(col0 + j * 128, 128)], idx_v.at[:, j, :],
            isem).wait()
    inv_a = scale_v[pl.ds(0, 16)]
    a = scale_v[pl.ds(16, 16)]

    gsems = (gsem0, gsem1)
    osems = (osem0, osem1)

    def supsub(t, slot):
        s = lax.div(t, BPW)
        j = lax.rem(t, BPW)

        def body(i, _):
            v = idx_v[s, j, pl.ds(i * 16, 16)]
            sup_v[slot, pl.ds(i * 16, 16)] = lax.shift_right_logical(v, 3)
            sub_v[slot, pl.ds(i * 16, 16)] = (v & 7) * 16
            return 0
        lax.fori_loop(0, 8, body, 0, unroll=4)

    def start_gather(slot):
        pltpu.async_copy(
            w_hbm.at[sup_v.at[slot]], super_v.at[slot], gsems[slot])

    def wait_gather(slot):
        pltpu.make_async_copy(
            w_hbm.at[sup_v.at[slot]], super_v.at[slot], gsems[slot]).wait()

    def start_out(t, slot):
        s = lax.div(t, BPW)
        j = lax.rem(t, BPW)
        for eb in range(2):
            pltpu.async_copy(
                res_v.at[slot, pl.ds(eb * 8, 8)],
                out_hbm.at[s, eb, wid * BPW + j], osems[slot])

    def wait_out(t, slot):
        s = lax.div(t, BPW)
        j = lax.rem(t, BPW)
        for eb in range(2):
            pltpu.make_async_copy(
                res_v.at[slot, pl.ds(eb * 8, 8)],
                out_hbm.at[s, eb, wid * BPW + j], osems[slot]).wait()

    iota = lax.iota(jnp.int32, 16)

    def extract_quant(slot):
        def body(i16, _):
            local = i16 * 16 + iota
            s16 = sub_v[slot, pl.ds(i16 * 16, 16)]
            for k in range(EMB_DIM):
                v = plsc.load_gather(super_v.at[slot], [local, s16 + k])
                r = (v * inv_a + MAGIC) - MAGIC
                r = jnp.minimum(jnp.maximum(r, QLOW), QHIGH)
                res_v[slot, k, pl.ds(i16 * 16, 16)] = r * a
            return 0
        lax.fori_loop(0, 8, body, 0)

    supsub(0, 0)
    start_gather(0)
    supsub(1, 1)
    start_gather(1)

    def pair(p, _):
        for slot in (0, 1):
            t = 2 * p + slot
            wait_gather(slot)

            @pl.when(p >= 1)
            def _():
                wait_out(t - 2, slot)

            extract_quant(slot)

            @pl.when(p < TILES_PW // 2 - 1)
            def _():
                supsub(t + 2, slot)
                start_gather(slot)

            start_out(t, slot)
        return 0

    lax.fori_loop(0, TILES_PW // 2, pair, 0)
    wait_out(TILES_PW - 2, 0)
    wait_out(TILES_PW - 1, 1)


def kernel(x, weight, alpha):
    a = jnp.abs(alpha).astype(jnp.float32) + 1e-10
    scale = jnp.concatenate([jnp.full((16,), 1.0 / a, jnp.float32),
                             jnp.full((16,), a, jnp.float32)])
    xt = x.T.astype(jnp.int32)
    w2 = _transpose_table(weight.T)
    out5 = _lsq_lookup(xt, w2, scale)
    return out5.transpose(2, 4, 0, 1, 3).reshape(BATCH, SLOTS, EMB_DIM)


# speedup vs baseline: 2.1378x; 1.3811x over previous
import functools

import jax
import jax.numpy as jnp
from jax import lax
from jax.experimental import pallas as pl
from jax.experimental.pallas import tpu as pltpu
from jax.experimental.pallas import tpu_sc as plsc

EMB_DIM = 16
VOCAB = 1000000
SUP_ROWS = VOCAB * EMB_DIM // 128
BATCH = 16384
SLOTS = 26
NUM_WORKERS = 32
NBLK = BATCH // 128
BPW = NBLK // NUM_WORKERS
TILES_PW = SLOTS * BPW
MAGIC = 12582912.0
QLOW = -128.0
QHIGH = 127.0

_mesh = plsc.VectorSubcoreMesh(core_axis_name="c", subcore_axis_name="s")

NTILE_T = VOCAB // 128
TAIL_COLS = VOCAB - NTILE_T * 128
TITER = (NTILE_T + NUM_WORKERS - 1) // NUM_WORKERS


@functools.partial(
    pl.kernel,
    out_type=jax.ShapeDtypeStruct((SUP_ROWS, 128), jnp.float32),
    mesh=_mesh,
    scratch_types=[
        pltpu.VMEM((2, EMB_DIM, 128), jnp.float32),
        pltpu.VMEM((2, EMB_DIM, 128), jnp.float32),
        pltpu.VMEM((EMB_DIM, TAIL_COLS), jnp.float32),
        pltpu.SemaphoreType.DMA,
        pltpu.SemaphoreType.DMA,
        pltpu.SemaphoreType.DMA,
        pltpu.SemaphoreType.DMA,
    ],
    compiler_params=pltpu.CompilerParams(needs_layout_passes=False),
)
def _transpose_table(wt_hbm, w2_hbm, tin_v, tout_v, tail_v, l0, l1, s0, s1):
    wid = lax.axis_index("s") * 2 + lax.axis_index("c")
    iota = lax.iota(jnp.int32, 16)
    lsems = (l0, l1)
    ssems = (s0, s1)
    cb = [(iota + j) & 7 for j in range(8)]
    dc = [((iota + j) & 7) * 16 + iota for j in range(8)]

    def c0_of(k):
        return pl.multiple_of((wid + NUM_WORKERS * k) * 128, 128)

    def sp0_of(k):
        return pl.multiple_of((wid + NUM_WORKERS * k) * 16, 16)

    def valid(k):
        return (wid + NUM_WORKERS * k) < NTILE_T

    def start_load(k, slot):
        pltpu.async_copy(wt_hbm.at[:, pl.ds(c0_of(k), 128)],
                         tin_v.at[slot], lsems[slot])

    def wait_load(k, slot):
        pltpu.make_async_copy(wt_hbm.at[:, pl.ds(c0_of(k), 128)],
                              tin_v.at[slot], lsems[slot]).wait()

    def start_store(k, slot):
        pltpu.async_copy(tout_v.at[slot],
                         w2_hbm.at[pl.ds(sp0_of(k), 16)], ssems[slot])

    def wait_store(k, slot):
        pltpu.make_async_copy(tout_v.at[slot],
                              w2_hbm.at[pl.ds(sp0_of(k), 16)],
                              ssems[slot]).wait()

    @pl.when(valid(0))
    def _():
        start_load(0, 0)

    @pl.when(valid(1))
    def _():
        start_load(1, 1)

    def pair(p, _):
        for slot in (0, 1):
            k = 2 * p + slot

            @pl.when(valid(k))
            def _():
                wait_load(k, slot)

                @pl.when(k >= 2)
                def _():
                    wait_store(k - 2, slot)

                for sp in range(16):
                    for j in range(8):
                        v = plsc.load_gather(tin_v.at[slot],
                                             [iota, sp * 8 + cb[j]])
                        plsc.store_scatter(tout_v.at[slot, sp], [dc[j]], v)

                @pl.when(valid(k + 2))
                def _():
                    start_load(k + 2, slot)

                start_store(k, slot)
        return 0

    lax.fori_loop(0, (TITER + 1) // 2, pair, 0)

    @pl.when(valid(TITER - 2))
    def _():
        wait_store(TITER - 2, (TITER - 2) % 2)

    @pl.when(valid(TITER - 1))
    def _():
        wait_store(TITER - 1, (TITER - 1) % 2)

    @pl.when(wid == 0)
    def _():
        pltpu.sync_copy(wt_hbm.at[:, pl.ds(NTILE_T * 128, TAIL_COLS)],
                        tail_v)
        for sp in range(TAIL_COLS // 8):
            for j in range(8):
                v = plsc.load_gather(tail_v, [iota, sp * 8 + cb[j]])
                plsc.store_scatter(tout_v.at[0, sp], [dc[j]], v)
        pltpu.sync_copy(tout_v.at[0, pl.ds(0, TAIL_COLS // 8)],
                        w2_hbm.at[pl.ds(NTILE_T * 16, TAIL_COLS // 8)])


@functools.partial(
    pl.kernel,
    out_type=jax.ShapeDtypeStruct((SLOTS, 2, NBLK, 8, 128), jnp.float32),
    mesh=_mesh,
    scratch_types=[
        pltpu.VMEM((SLOTS, BPW, 128), jnp.int32),
        pltpu.VMEM((2, 128), jnp.int32),
        pltpu.VMEM((2, 128), jnp.int32),
        pltpu.VMEM((2, 128, 128), jnp.float32),
        pltpu.VMEM((2, EMB_DIM, 128), jnp.float32),
        pltpu.VMEM((32,), jnp.float32),
        pltpu.SemaphoreType.DMA,
        pltpu.SemaphoreType.DMA,
        pltpu.SemaphoreType.DMA,
        pltpu.SemaphoreType.DMA,
        pltpu.SemaphoreType.DMA,
    ],
    compiler_params=pltpu.CompilerParams(needs_layout_passes=False),
)
def _lsq_lookup(xt_hbm, w_hbm, scale_hbm, out_hbm,
                idx_v, sup_v, sub_v, super_v, res_v, scale_v,
                isem, gsem0, gsem1, osem0, osem1):
    wid = lax.axis_index("s") * 2 + lax.axis_index("c")
    col0 = wid * (BPW * 128)

    pltpu.sync_copy(scale_hbm, scale_v)
    for j in range(BPW):
        pltpu.async_copy(
            xt_hbm.at[:, pl.ds(col0 + j * 128, 128)], idx_v.at[:, j, :],
            isem)
    for j in range(BPW):
        pltpu.make_async_copy(
            xt_hbm.at[:, pl.ds(col0 + j * 128, 128)], idx_v.at[:, j, :],
            isem).wait()
    inv_a = scale_v[pl.ds(0, 16)]
    a = scale_v[pl.ds(16, 16)]

    gsems = (gsem0, gsem1)
    osems = (osem0, osem1)

    def supsub(t, slot):
        s = lax.div(t, BPW)
        j = lax.rem(t, BPW)

        def body(i, _):
            v = idx_v[s, j, pl.ds(i * 16, 16)]
            sup_v[slot, pl.ds(i * 16, 16)] = lax.shift_right_logical(v, 3)
            sub_v[slot, pl.ds(i * 16, 16)] = (v & 7) * 16
            return 0
        lax.fori_loop(0, 8, body, 0, unroll=4)

    def start_gather(slot):
        pltpu.async_copy(
            w_hbm.at[sup_v.at[slot]], super_v.at[slot], gsems[slot])

    def wait_gather(slot):
        pltpu.make_async_copy(
            w_hbm.at[sup_v.at[slot]], super_v.at[slot], gsems[slot]).wait()

    def start_out(t, slot):
        s = lax.div(t, BPW)
        j = lax.rem(t, BPW)
        for eb in range(2):
            pltpu.async_copy(
                res_v.at[slot, pl.ds(eb * 8, 8)],
                out_hbm.at[s, eb, wid * BPW + j], osems[slot])

    def wait_out(t, slot):
        s = lax.div(t, BPW)
        j = lax.rem(t, BPW)
        for eb in range(2):
            pltpu.make_async_copy(
                res_v.at[slot, pl.ds(eb * 8, 8)],
                out_hbm.at[s, eb, wid * BPW + j], osems[slot]).wait()

    iota = lax.iota(jnp.int32, 16)

    def extract_quant(slot):
        def body(i16, _):
            local = i16 * 16 + iota
            s16 = sub_v[slot, pl.ds(i16 * 16, 16)]
            for k in range(EMB_DIM):
                v = plsc.load_gather(super_v.at[slot], [local, s16 + k])
                r = (v * inv_a + MAGIC) - MAGIC
                r = jnp.minimum(jnp.maximum(r, QLOW), QHIGH)
                res_v[slot, k, pl.ds(i16 * 16, 16)] = r * a
            return 0
        lax.fori_loop(0, 8, body, 0)

    supsub(0, 0)
    start_gather(0)
    supsub(1, 1)
    start_gather(1)

    def pair(p, _):
        for slot in (0, 1):
            t = 2 * p + slot
            wait_gather(slot)

            @pl.when(p >= 1)
            def _():
                wait_out(t - 2, slot)

            extract_quant(slot)

            @pl.when(p < TILES_PW // 2 - 1)
            def _():
                supsub(t + 2, slot)
                start_gather(slot)

            start_out(t, slot)
        return 0

    lax.fori_loop(0, TILES_PW // 2, pair, 0)
    wait_out(TILES_PW - 2, 0)
    wait_out(TILES_PW - 1, 1)


def kernel(x, weight, alpha):
    a = jnp.abs(alpha).astype(jnp.float32) + 1e-10
    scale = jnp.concatenate([jnp.full((16,), 1.0 / a, jnp.float32),
                             jnp.full((16,), a, jnp.float32)])
    xt = x.T.astype(jnp.int32)
    w2 = _transpose_table(weight.T)
    out5 = _lsq_lookup(xt, w2, scale)
    return out5.transpose(2, 4, 0, 1, 3).reshape(BATCH, SLOTS, EMB_DIM)


# speedup vs baseline: 3.0936x; 1.4471x over previous
import functools

import jax
import jax.numpy as jnp
from jax import lax
from jax.experimental import pallas as pl
from jax.experimental.pallas import tpu as pltpu
from jax.experimental.pallas import tpu_sc as plsc

EMB_DIM = 16
VOCAB = 1000000
SUP_ROWS = VOCAB * EMB_DIM // 128
BATCH = 16384
SLOTS = 26
NUM_WORKERS = 32
NBLK = BATCH // 128
BPW = NBLK // NUM_WORKERS
TILES_PW = SLOTS * BPW
MAGIC = 12582912.0
QLOW = -128.0
QHIGH = 127.0

_mesh = plsc.VectorSubcoreMesh(core_axis_name="c", subcore_axis_name="s")

TCOL = 512
NTILE_T = VOCAB // TCOL
TAIL_COLS = VOCAB - NTILE_T * TCOL
TITER = (NTILE_T + NUM_WORKERS - 1) // NUM_WORKERS


@functools.partial(
    pl.kernel,
    out_type=jax.ShapeDtypeStruct((SUP_ROWS, 128), jnp.float32),
    mesh=_mesh,
    scratch_types=[
        pltpu.VMEM((2, EMB_DIM, TCOL), jnp.float32),
        pltpu.VMEM((2, TCOL // 8, 128), jnp.float32),
        pltpu.VMEM((EMB_DIM, TAIL_COLS), jnp.float32),
        pltpu.SemaphoreType.DMA,
        pltpu.SemaphoreType.DMA,
        pltpu.SemaphoreType.DMA,
        pltpu.SemaphoreType.DMA,
    ],
    compiler_params=pltpu.CompilerParams(needs_layout_passes=False),
)
def _transpose_table(wt_hbm, w2_hbm, tin_v, tout_v, tail_v, l0, l1, s0, s1):
    wid = lax.axis_index("s") * 2 + lax.axis_index("c")
    iota = lax.iota(jnp.int32, 16)
    lsems = (l0, l1)
    ssems = (s0, s1)
    f16 = [(iota + j) & 15 for j in range(16)]
    rowo = [((iota + j) & 15) >> 3 for j in range(16)]
    colw = [(((iota + j) & 15) & 7) * 16 + iota for j in range(16)]

    def c0_of(k):
        return pl.multiple_of((wid + NUM_WORKERS * k) * TCOL, TCOL)

    def sp0_of(k):
        return pl.multiple_of((wid + NUM_WORKERS * k) * (TCOL // 8),
                              TCOL // 8)

    def valid(k):
        return (wid + NUM_WORKERS * k) < NTILE_T

    def start_load(k, slot):
        pltpu.async_copy(wt_hbm.at[:, pl.ds(c0_of(k), TCOL)],
                         tin_v.at[slot], lsems[slot])

    def wait_load(k, slot):
        pltpu.make_async_copy(wt_hbm.at[:, pl.ds(c0_of(k), TCOL)],
                              tin_v.at[slot], lsems[slot]).wait()

    def start_store(k, slot):
        pltpu.async_copy(tout_v.at[slot],
                         w2_hbm.at[pl.ds(sp0_of(k), TCOL // 8)],
                         ssems[slot])

    def wait_store(k, slot):
        pltpu.make_async_copy(tout_v.at[slot],
                              w2_hbm.at[pl.ds(sp0_of(k), TCOL // 8)],
                              ssems[slot]).wait()

    @pl.when(valid(0))
    def _():
        start_load(0, 0)

    @pl.when(valid(1))
    def _():
        start_load(1, 1)

    def pair(p, _):
        for slot in (0, 1):
            k = 2 * p + slot

            @pl.when(valid(k))
            def _():
                wait_load(k, slot)

                @pl.when(k >= 2)
                def _():
                    wait_store(k - 2, slot)

                def shuffle(sp2, _):
                    for j in range(16):
                        v = plsc.load_gather(tin_v.at[slot],
                                             [iota, sp2 * 16 + f16[j]])
                        plsc.store_scatter(tout_v.at[slot],
                                           [sp2 * 2 + rowo[j], colw[j]], v)
                    return 0
                lax.fori_loop(0, TCOL // 16, shuffle, 0, unroll=2)

                @pl.when(valid(k + 2))
                def _():
                    start_load(k + 2, slot)

                start_store(k, slot)
        return 0

    lax.fori_loop(0, (TITER + 1) // 2, pair, 0)

    @pl.when(valid(TITER - 2))
    def _():
        wait_store(TITER - 2, (TITER - 2) % 2)

    @pl.when(valid(TITER - 1))
    def _():
        wait_store(TITER - 1, (TITER - 1) % 2)

    @pl.when(wid == 0)
    def _():
        pltpu.sync_copy(wt_hbm.at[:, pl.ds(NTILE_T * TCOL, TAIL_COLS)],
                        tail_v)
        for sp2 in range(TAIL_COLS // 16):
            for j in range(16):
                v = plsc.load_gather(tail_v, [iota, sp2 * 16 + f16[j]])
                plsc.store_scatter(tout_v.at[0],
                                   [sp2 * 2 + rowo[j], colw[j]], v)
        pltpu.sync_copy(tout_v.at[0, pl.ds(0, TAIL_COLS // 8)],
                        w2_hbm.at[pl.ds(NTILE_T * (TCOL // 8),
                                        TAIL_COLS // 8)])


@functools.partial(
    pl.kernel,
    out_type=jax.ShapeDtypeStruct((SLOTS, 2, NBLK, 8, 128), jnp.float32),
    mesh=_mesh,
    scratch_types=[
        pltpu.VMEM((SLOTS, BPW, 128), jnp.int32),
        pltpu.VMEM((2, 128), jnp.int32),
        pltpu.VMEM((2, 128), jnp.int32),
        pltpu.VMEM((2, 128, 128), jnp.float32),
        pltpu.VMEM((2, EMB_DIM, 128), jnp.float32),
        pltpu.VMEM((32,), jnp.float32),
        pltpu.SemaphoreType.DMA,
        pltpu.SemaphoreType.DMA,
        pltpu.SemaphoreType.DMA,
        pltpu.SemaphoreType.DMA,
        pltpu.SemaphoreType.DMA,
    ],
    compiler_params=pltpu.CompilerParams(needs_layout_passes=False),
)
def _lsq_lookup(xt_hbm, w_hbm, scale_hbm, out_hbm,
                idx_v, sup_v, sub_v, super_v, res_v, scale_v,
                isem, gsem0, gsem1, osem0, osem1):
    wid = lax.axis_index("s") * 2 + lax.axis_index("c")
    col0 = wid * (BPW * 128)

    pltpu.sync_copy(scale_hbm, scale_v)
    for j in range(BPW):
        pltpu.async_copy(
            xt_hbm.at[:, pl.ds(col0 + j * 128, 128)], idx_v.at[:, j, :],
            isem)
    for j in range(BPW):
        pltpu.make_async_copy(
            xt_hbm.at[:, pl.ds(col0 + j * 128, 128)], idx_v.at[:, j, :],
            isem).wait()
    inv_a = scale_v[pl.ds(0, 16)]
    a = scale_v[pl.ds(16, 16)]

    gsems = (gsem0, gsem1)
    osems = (osem0, osem1)

    def supsub(t, slot):
        s = lax.div(t, BPW)
        j = lax.rem(t, BPW)

        def body(i, _):
            v = idx_v[s, j, pl.ds(i * 16, 16)]
            sup_v[slot, pl.ds(i * 16, 16)] = lax.shift_right_logical(v, 3)
            sub_v[slot, pl.ds(i * 16, 16)] = (v & 7) * 16
            return 0
        lax.fori_loop(0, 8, body, 0, unroll=4)

    def start_gather(slot):
        pltpu.async_copy(
            w_hbm.at[sup_v.at[slot]], super_v.at[slot], gsems[slot])

    def wait_gather(slot):
        pltpu.make_async_copy(
            w_hbm.at[sup_v.at[slot]], super_v.at[slot], gsems[slot]).wait()

    def start_out(t, slot):
        s = lax.div(t, BPW)
        j = lax.rem(t, BPW)
        for eb in range(2):
            pltpu.async_copy(
                res_v.at[slot, pl.ds(eb * 8, 8)],
                out_hbm.at[s, eb, wid * BPW + j], osems[slot])

    def wait_out(t, slot):
        s = lax.div(t, BPW)
        j = lax.rem(t, BPW)
        for eb in range(2):
            pltpu.make_async_copy(
                res_v.at[slot, pl.ds(eb * 8, 8)],
                out_hbm.at[s, eb, wid * BPW + j], osems[slot]).wait()

    iota = lax.iota(jnp.int32, 16)
    feat = [(iota + k) & 15 for k in range(EMB_DIM)]

    def extract_quant(slot):
        def body(i16, _):
            local = i16 * 16 + iota
            s16 = sub_v[slot, pl.ds(i16 * 16, 16)]
            for k in range(EMB_DIM):
                v = plsc.load_gather(super_v.at[slot],
                                     [local, s16 + feat[k]])
                r = (v * inv_a + MAGIC) - MAGIC
                r = jnp.minimum(jnp.maximum(r, QLOW), QHIGH)
                plsc.store_scatter(res_v.at[slot], [feat[k], local], r * a)
            return 0
        lax.fori_loop(0, 8, body, 0)

    supsub(0, 0)
    start_gather(0)
    supsub(1, 1)
    start_gather(1)

    def pair(p, _):
        for slot in (0, 1):
            t = 2 * p + slot
            wait_gather(slot)

            @pl.when(p >= 1)
            def _():
                wait_out(t - 2, slot)

            extract_quant(slot)

            @pl.when(p < TILES_PW // 2 - 1)
            def _():
                supsub(t + 2, slot)
                start_gather(slot)

            start_out(t, slot)
        return 0

    lax.fori_loop(0, TILES_PW // 2, pair, 0)
    wait_out(TILES_PW - 2, 0)
    wait_out(TILES_PW - 1, 1)


def kernel(x, weight, alpha):
    a = jnp.abs(alpha).astype(jnp.float32) + 1e-10
    scale = jnp.concatenate([jnp.full((16,), 1.0 / a, jnp.float32),
                             jnp.full((16,), a, jnp.float32)])
    xt = x.T.astype(jnp.int32)
    w2 = _transpose_table(weight.T)
    out5 = _lsq_lookup(xt, w2, scale)
    return out5.transpose(2, 4, 0, 1, 3).reshape(BATCH, SLOTS, EMB_DIM)


# speedup vs baseline: 3.1476x; 1.0174x over previous
import functools

import jax
import jax.numpy as jnp
from jax import lax
from jax.experimental import pallas as pl
from jax.experimental.pallas import tpu as pltpu
from jax.experimental.pallas import tpu_sc as plsc

EMB_DIM = 16
VOCAB = 1000000
SUP_ROWS = VOCAB * EMB_DIM // 128
BATCH = 16384
SLOTS = 26
NUM_WORKERS = 32
NBLK = BATCH // 128
BPW = NBLK // NUM_WORKERS
TILES_PW = SLOTS * BPW // 2
MAGIC = 12582912.0
QLOW = -128.0
QHIGH = 127.0

_mesh = plsc.VectorSubcoreMesh(core_axis_name="c", subcore_axis_name="s")

TCOL = 896
NTILE_T = VOCAB // TCOL
TAIL_COLS = VOCAB - NTILE_T * TCOL
TITER = (NTILE_T + NUM_WORKERS - 1) // NUM_WORKERS


@functools.partial(
    pl.kernel,
    out_type=jax.ShapeDtypeStruct((SUP_ROWS, 128), jnp.float32),
    mesh=_mesh,
    scratch_types=[
        pltpu.VMEM((2, EMB_DIM, TCOL), jnp.float32),
        pltpu.VMEM((2, TCOL // 8, 128), jnp.float32),
        pltpu.VMEM((EMB_DIM, TAIL_COLS), jnp.float32),
        pltpu.SemaphoreType.DMA,
        pltpu.SemaphoreType.DMA,
        pltpu.SemaphoreType.DMA,
        pltpu.SemaphoreType.DMA,
    ],
    compiler_params=pltpu.CompilerParams(needs_layout_passes=False),
)
def _transpose_table(wt_hbm, w2_hbm, tin_v, tout_v, tail_v, l0, l1, s0, s1):
    wid = lax.axis_index("s") * 2 + lax.axis_index("c")
    iota = lax.iota(jnp.int32, 16)
    lsems = (l0, l1)
    ssems = (s0, s1)
    f16 = [(iota + j) & 15 for j in range(16)]
    rowo = [((iota + j) & 15) >> 3 for j in range(16)]
    colw = [(((iota + j) & 15) & 7) * 16 + iota for j in range(16)]

    def c0_of(k):
        return pl.multiple_of((wid + NUM_WORKERS * k) * TCOL, TCOL)

    def sp0_of(k):
        return pl.multiple_of((wid + NUM_WORKERS * k) * (TCOL // 8),
                              TCOL // 8)

    def valid(k):
        return (wid + NUM_WORKERS * k) < NTILE_T

    def start_load(k, slot):
        pltpu.async_copy(wt_hbm.at[:, pl.ds(c0_of(k), TCOL)],
                         tin_v.at[slot], lsems[slot])

    def wait_load(k, slot):
        pltpu.make_async_copy(wt_hbm.at[:, pl.ds(c0_of(k), TCOL)],
                              tin_v.at[slot], lsems[slot]).wait()

    def start_store(k, slot):
        pltpu.async_copy(tout_v.at[slot],
                         w2_hbm.at[pl.ds(sp0_of(k), TCOL // 8)],
                         ssems[slot])

    def wait_store(k, slot):
        pltpu.make_async_copy(tout_v.at[slot],
                              w2_hbm.at[pl.ds(sp0_of(k), TCOL // 8)],
                              ssems[slot]).wait()

    @pl.when(valid(0))
    def _():
        start_load(0, 0)

    @pl.when(valid(1))
    def _():
        start_load(1, 1)

    def pair(p, _):
        for slot in (0, 1):
            k = 2 * p + slot

            @pl.when(valid(k))
            def _():
                wait_load(k, slot)

                @pl.when(k >= 2)
                def _():
                    wait_store(k - 2, slot)

                def shuffle(sp2, _):
                    for j in range(16):
                        v = plsc.load_gather(tin_v.at[slot],
                                             [iota, sp2 * 16 + f16[j]])
                        plsc.store_scatter(tout_v.at[slot],
                                           [sp2 * 2 + rowo[j], colw[j]], v)
                    return 0
                lax.fori_loop(0, TCOL // 16, shuffle, 0, unroll=2)

                @pl.when(valid(k + 2))
                def _():
                    start_load(k + 2, slot)

                start_store(k, slot)
        return 0

    lax.fori_loop(0, (TITER + 1) // 2, pair, 0)

    @pl.when(valid(TITER - 2))
    def _():
        wait_store(TITER - 2, (TITER - 2) % 2)

    @pl.when(valid(TITER - 1))
    def _():
        wait_store(TITER - 1, (TITER - 1) % 2)

    @pl.when(wid == 0)
    def _():
        pltpu.sync_copy(wt_hbm.at[:, pl.ds(NTILE_T * TCOL, TAIL_COLS)],
                        tail_v)
        for sp2 in range(TAIL_COLS // 16):
            for j in range(16):
                v = plsc.load_gather(tail_v, [iota, sp2 * 16 + f16[j]])
                plsc.store_scatter(tout_v.at[0],
                                   [sp2 * 2 + rowo[j], colw[j]], v)
        pltpu.sync_copy(tout_v.at[0, pl.ds(0, TAIL_COLS // 8)],
                        w2_hbm.at[pl.ds(NTILE_T * (TCOL // 8),
                                        TAIL_COLS // 8)])


@functools.partial(
    pl.kernel,
    out_type=jax.ShapeDtypeStruct((SLOTS, 2, NBLK, 8, 128), jnp.float32),
    mesh=_mesh,
    scratch_types=[
        pltpu.VMEM((SLOTS, BPW, 128), jnp.int32),
        pltpu.VMEM((2, 2, 128), jnp.int32),
        pltpu.VMEM((2, 256), jnp.int32),
        pltpu.VMEM((2, 256, 128), jnp.float32),
        pltpu.VMEM((2, EMB_DIM, 256), jnp.float32),
        pltpu.VMEM((32,), jnp.float32),
        pltpu.SemaphoreType.DMA,
        pltpu.SemaphoreType.DMA,
        pltpu.SemaphoreType.DMA,
        pltpu.SemaphoreType.DMA,
        pltpu.SemaphoreType.DMA,
    ],
    compiler_params=pltpu.CompilerParams(needs_layout_passes=False),
)
def _lsq_lookup(xt_hbm, w_hbm, scale_hbm, out_hbm,
                idx_v, sup_v, sub_v, super_v, res_v, scale_v,
                isem, gsem0, gsem1, osem0, osem1):
    wid = lax.axis_index("s") * 2 + lax.axis_index("c")
    col0 = wid * (BPW * 128)

    pltpu.sync_copy(scale_hbm, scale_v)
    for j in range(BPW):
        pltpu.async_copy(
            xt_hbm.at[:, pl.ds(col0 + j * 128, 128)], idx_v.at[:, j, :],
            isem)
    for j in range(BPW):
        pltpu.make_async_copy(
            xt_hbm.at[:, pl.ds(col0 + j * 128, 128)], idx_v.at[:, j, :],
            isem).wait()
    inv_a = scale_v[pl.ds(0, 16)]
    a = scale_v[pl.ds(16, 16)]

    gsems = (gsem0, gsem1)
    osems = (osem0, osem1)

    def supsub(t, slot):
        s = lax.div(t, 2)
        jp = lax.rem(t, 2)
        for jj in range(2):
            def body(i, _):
                v = idx_v[s, jp * 2 + jj, pl.ds(i * 16, 16)]
                sup_v[slot, jj, pl.ds(i * 16, 16)] = (
                    lax.shift_right_logical(v, 3))
                sub_v[slot, pl.ds(jj * 128 + i * 16, 16)] = (v & 7) * 16
                return 0
            lax.fori_loop(0, 8, body, 0, unroll=4)

    def start_gather(slot):
        for jj in range(2):
            pltpu.async_copy(
                w_hbm.at[sup_v.at[slot, jj]],
                super_v.at[slot, pl.ds(jj * 128, 128)], gsems[slot])

    def wait_gather(slot):
        for jj in range(2):
            pltpu.make_async_copy(
                w_hbm.at[sup_v.at[slot, jj]],
                super_v.at[slot, pl.ds(jj * 128, 128)], gsems[slot]).wait()

    def start_out(t, slot):
        s = lax.div(t, 2)
        jp = lax.rem(t, 2)
        for eb in range(2):
            for jj in range(2):
                pltpu.async_copy(
                    res_v.at[slot, pl.ds(eb * 8, 8), pl.ds(jj * 128, 128)],
                    out_hbm.at[s, eb, wid * BPW + jp * 2 + jj],
                    osems[slot])

    def wait_out(t, slot):
        s = lax.div(t, 2)
        jp = lax.rem(t, 2)
        for eb in range(2):
            for jj in range(2):
                pltpu.make_async_copy(
                    res_v.at[slot, pl.ds(eb * 8, 8), pl.ds(jj * 128, 128)],
                    out_hbm.at[s, eb, wid * BPW + jp * 2 + jj],
                    osems[slot]).wait()

    iota = lax.iota(jnp.int32, 16)
    feat = [(iota + k) & 15 for k in range(EMB_DIM)]

    def extract_quant(slot):
        def body(i16, _):
            local = i16 * 16 + iota
            s16 = sub_v[slot, pl.ds(i16 * 16, 16)]
            for k in range(EMB_DIM):
                v = plsc.load_gather(super_v.at[slot],
                                     [local, s16 + feat[k]])
                r = (v * inv_a + MAGIC) - MAGIC
                r = jnp.minimum(jnp.maximum(r, QLOW), QHIGH)
                plsc.store_scatter(res_v.at[slot], [feat[k], local], r * a)
            return 0
        lax.fori_loop(0, 16, body, 0)

    supsub(0, 0)
    start_gather(0)
    supsub(1, 1)
    start_gather(1)

    def pair(p, _):
        for slot in (0, 1):
            t = 2 * p + slot
            wait_gather(slot)

            @pl.when(p >= 1)
            def _():
                wait_out(t - 2, slot)

            extract_quant(slot)

            @pl.when(p < TILES_PW // 2 - 1)
            def _():
                supsub(t + 2, slot)
                start_gather(slot)

            start_out(t, slot)
        return 0

    lax.fori_loop(0, TILES_PW // 2, pair, 0)
    wait_out(TILES_PW - 2, 0)
    wait_out(TILES_PW - 1, 1)


def kernel(x, weight, alpha):
    a = jnp.abs(alpha).astype(jnp.float32) + 1e-10
    scale = jnp.concatenate([jnp.full((16,), 1.0 / a, jnp.float32),
                             jnp.full((16,), a, jnp.float32)])
    xt = x.T.astype(jnp.int32)
    w2 = _transpose_table(weight.T)
    out5 = _lsq_lookup(xt, w2, scale)
    return out5.transpose(2, 4, 0, 1, 3).reshape(BATCH, SLOTS, EMB_DIM)
